# baseline (speedup 1.0000x reference)
import functools

import jax
import jax.numpy as jnp
from jax import lax
from jax.experimental import pallas as pl
from jax.experimental.pallas import tpu as pltpu
from jax.experimental.pallas import tpu_sc as plsc

_N_BINS = 8
_P = 1024
_D = 64
_T = 4608
_NW = 32
_TPW = _T // _NW
_CW = 16
_NCHUNK = _TPW // _CW


def _match_kernel(edges_ref, x_ref, pat_ref, val_ref):
    xb = x_ref[...]
    pb = pat_ref[...]

    qx = jnp.zeros(xb.shape, jnp.float32)
    qp = jnp.zeros(pb.shape, jnp.float32)
    for i in range(7):
        e = edges_ref[i]
        qx = qx + (xb > e).astype(jnp.float32)
        qp = qp + (pb > e).astype(jnp.float32)

    acc = jnp.zeros((xb.shape[0], _P), jnp.float32)
    for b in range(_N_BINS):
        a = (qx == b).astype(jnp.bfloat16)
        p1 = (qp == b).astype(jnp.bfloat16)
        acc = acc + lax.dot_general(
            a, p1, (((1,), (1,)), ((), ())),
            preferred_element_type=jnp.float32)

    counts = acc.astype(jnp.int32)
    rev = (_P - 1) - lax.broadcasted_iota(jnp.int32, acc.shape, 1)
    val_ref[...] = counts * _P + rev


def _sc_argmax(val_hbm, best_hbm, score_hbm, buf, tmp, bb, sb):
    wid = lax.axis_index("s") * 2 + lax.axis_index("c")
    tbase = wid * _TPW
    lane = lax.iota(jnp.int32, 16)

    def chunk_body(c, _):
        base = tbase + c * _CW
        for t in range(_CW):
            pltpu.sync_copy(val_hbm.at[base + t], buf.at[pl.ds(t * _P, _P)])

        m_all = jnp.full((16,), -1, jnp.int32)
        for t in range(_CW):
            def body(g, m, t=t):
                off = pl.multiple_of(t * _P + g * 16, 16)
                return jnp.maximum(m, buf[pl.ds(off, 16)])
            m16 = lax.fori_loop(0, _P // 16, body,
                                jnp.full((16,), -1, jnp.int32))
            s = m16[0]
            for i in range(1, 16):
                s = jnp.maximum(s, m16[i])
            m_all = jnp.where(lane == t, s, m_all)
        bb[pl.ds(c * _CW, _CW)] = (_P - 1) - (m_all & (_P - 1))
        sb[pl.ds(c * _CW, _CW)] = (m_all >> 10).astype(jnp.float32) * (1.0 / _D)
        return 0

    lax.fori_loop(0, _NCHUNK, chunk_body, 0)
    pltpu.sync_copy(bb, best_hbm.at[pl.ds(tbase, _TPW)])
    pltpu.sync_copy(sb, score_hbm.at[pl.ds(tbase, _TPW)])


@functools.partial(
    pl.kernel,
    mesh=plsc.VectorSubcoreMesh(core_axis_name="c", subcore_axis_name="s"),
    out_type=[
        jax.ShapeDtypeStruct((_T,), jnp.int32),
        jax.ShapeDtypeStruct((_T,), jnp.float32),
    ],
    scratch_types=[
        pltpu.VMEM((_CW * _P,), jnp.int32),
        pltpu.VMEM((16,), jnp.int32),
        pltpu.VMEM((_TPW,), jnp.int32),
        pltpu.VMEM((_TPW,), jnp.float32),
    ],
)
def _sc_argmax_call(val_hbm, best_hbm, score_hbm, buf, tmp, bb, sb):
    _sc_argmax(val_hbm, best_hbm, score_hbm, buf, tmp, bb, sb)


def kernel(x, patterns, quantize_edges):
    B, S, D = x.shape
    t_tile = 512
    val = pl.pallas_call(
        _match_kernel,
        grid=(_T // t_tile,),
        in_specs=[
            pl.BlockSpec(memory_space=pltpu.SMEM),
            pl.BlockSpec((t_tile, D), lambda i: (i, 0)),
            pl.BlockSpec((_P, D), lambda i: (0, 0)),
        ],
        out_specs=pl.BlockSpec((t_tile, _P), lambda i: (i, 0)),
        out_shape=jax.ShapeDtypeStruct((B * S, _P), jnp.int32),
    )(quantize_edges, x.reshape(B * S, D), patterns)
    best, score = _sc_argmax_call(val)
    return best.reshape(B, S), score.reshape(B, S)


# speedup vs baseline: 2.2088x; 2.2088x over previous
import functools

import jax
import jax.numpy as jnp
from jax import lax
from jax.experimental import pallas as pl
from jax.experimental.pallas import tpu as pltpu
from jax.experimental.pallas import tpu_sc as plsc

_N_BINS = 8
_P = 1024
_D = 64
_T = 4608
_NW = 32
_TPW = _T // _NW
_CW = 16
_NCHUNK = _TPW // _CW


def _match_kernel(edges_ref, x_ref, pat_ref, val_ref):
    xb = x_ref[...]
    pb = pat_ref[...]

    qx = jnp.zeros(xb.shape, jnp.float32)
    qp = jnp.zeros(pb.shape, jnp.float32)
    for i in range(7):
        e = edges_ref[i]
        qx = qx + (xb > e).astype(jnp.float32)
        qp = qp + (pb > e).astype(jnp.float32)

    acc = jnp.zeros((xb.shape[0], _P), jnp.float32)
    for b in range(_N_BINS):
        a = (qx == b).astype(jnp.bfloat16)
        p1 = (qp == b).astype(jnp.bfloat16)
        acc = acc + lax.dot_general(
            a, p1, (((1,), (1,)), ((), ())),
            preferred_element_type=jnp.float32)

    counts = acc.astype(jnp.int32)
    rev = (_P - 1) - lax.broadcasted_iota(jnp.int32, acc.shape, 1)
    val_ref[...] = counts * _P + rev


def _sc_argmax(val_hbm, best_hbm, score_hbm, buf, tmp, bb, sb):
    wid = lax.axis_index("s") * 2 + lax.axis_index("c")
    tbase = wid * _TPW
    lane = lax.iota(jnp.int32, 16)

    def chunk_body(c, _):
        base = tbase + c * _CW
        pltpu.sync_copy(val_hbm.at[pl.ds(base, _CW), :], buf)

        m_all = jnp.full((16,), -1, jnp.int32)
        for t in range(_CW):
            def body(g, m, t=t):
                off = pl.multiple_of(g * 16, 16)
                return jnp.maximum(m, buf[t, pl.ds(off, 16)])
            m16 = lax.fori_loop(0, _P // 16, body,
                                jnp.full((16,), -1, jnp.int32),
                                unroll=4)
            s = m16[0]
            for i in range(1, 16):
                s = jnp.maximum(s, m16[i])
            m_all = jnp.where(lane == t, s, m_all)
        bb[pl.ds(c * _CW, _CW)] = (_P - 1) - (m_all & (_P - 1))
        sb[pl.ds(c * _CW, _CW)] = (m_all >> 10).astype(jnp.float32) * (1.0 / _D)
        return 0

    lax.fori_loop(0, _NCHUNK, chunk_body, 0)
    pltpu.sync_copy(bb, best_hbm.at[pl.ds(tbase, _TPW)])
    pltpu.sync_copy(sb, score_hbm.at[pl.ds(tbase, _TPW)])


@functools.partial(
    pl.kernel,
    mesh=plsc.VectorSubcoreMesh(core_axis_name="c", subcore_axis_name="s"),
    out_type=[
        jax.ShapeDtypeStruct((_T,), jnp.int32),
        jax.ShapeDtypeStruct((_T,), jnp.float32),
    ],
    scratch_types=[
        pltpu.VMEM((_CW, _P), jnp.int32),
        pltpu.VMEM((16,), jnp.int32),
        pltpu.VMEM((_TPW,), jnp.int32),
        pltpu.VMEM((_TPW,), jnp.float32),
    ],
)
def _sc_argmax_call(val_hbm, best_hbm, score_hbm, buf, tmp, bb, sb):
    _sc_argmax(val_hbm, best_hbm, score_hbm, buf, tmp, bb, sb)


def kernel(x, patterns, quantize_edges):
    B, S, D = x.shape
    t_tile = 512
    val = pl.pallas_call(
        _match_kernel,
        grid=(_T // t_tile,),
        in_specs=[
            pl.BlockSpec(memory_space=pltpu.SMEM),
            pl.BlockSpec((t_tile, D), lambda i: (i, 0)),
            pl.BlockSpec((_P, D), lambda i: (0, 0)),
        ],
        out_specs=pl.BlockSpec((t_tile, _P), lambda i: (i, 0)),
        out_shape=jax.ShapeDtypeStruct((B * S, _P), jnp.int32),
    )(quantize_edges, x.reshape(B * S, D), patterns)
    best, score = _sc_argmax_call(val)
    return best.reshape(B, S), score.reshape(B, S)
